# Initial kernel scaffold; baseline (speedup 1.0000x reference)
import jax
import jax.numpy as jnp
from jax.experimental import pallas as pl


def kernel(x, router_w, w1, w2):
    raise NotImplementedError("write your pallas kernel here")


# speedup vs baseline: 1.1104x; 1.1104x over previous
import functools

import jax
import jax.numpy as jnp
from jax.experimental import pallas as pl
from jax.experimental.pallas import tpu as pltpu

_AUX_W = 0.1
_Z_W = 0.001
_ROW_TILE = 256


def _router_kernel(x_ref, rw_ref, d_ref, gs_ref, loss_ref, s_ref, st_ref,
                   probs_ref, g_ref, *, n, e_total, cap, cap_pad):
    e = pl.program_id(0)

    @pl.when(e == 0)
    def _init():
        xl = x_ref[...]
        rw = rw_ref[...]
        logits = jax.lax.dot_general(
            xl, rw, (((1,), (1,)), ((), ())),
            preferred_element_type=jnp.float32)
        m = jnp.max(logits, axis=-1, keepdims=True)
        ex = jnp.exp(logits - m)
        sume = jnp.sum(ex, axis=-1, keepdims=True)
        probs = ex / sume
        probs_ref[...] = probs
        lse = m + jnp.log(sume)
        z_loss = _Z_W * jnp.sum(lse * lse) / n
        loss_ref[0, 0] = z_loss
        pmax = jnp.max(probs, axis=-1, keepdims=True)
        eiota = jax.lax.broadcasted_iota(jnp.int32, probs.shape, 1)
        eidx = jnp.min(jnp.where(probs == pmax, eiota, e_total),
                       axis=-1, keepdims=True)
        s = jnp.where(eiota == eidx, probs, -1.0)
        s_ref[...] = s
        st_ref[...] = jnp.transpose(s)

    st = st_ref[...]
    riota = jax.lax.broadcasted_iota(jnp.int32, st.shape, 0)
    sr = jnp.sum(jnp.where(riota == e, st, 0.0), axis=0, keepdims=True)
    cnt = jnp.zeros((), jnp.float32)
    psum = jnp.zeros((), jnp.float32)
    for t in range(n // _ROW_TILE):
        rows = pl.ds(t * _ROW_TILE, _ROW_TILE)
        s_tile = s_ref[rows, :]
        p_tile = probs_ref[rows, :]
        eiota = jax.lax.broadcasted_iota(jnp.int32, s_tile.shape, 1)
        sc = jnp.sum(jnp.where(eiota == e, s_tile, 0.0),
                     axis=1, keepdims=True)
        p = jnp.sum(jnp.where(eiota == e, p_tile, 0.0),
                    axis=1, keepdims=True)
        jiota = jax.lax.broadcasted_iota(jnp.int32, (_ROW_TILE, n), 1)
        iiota = (jax.lax.broadcasted_iota(jnp.int32, (_ROW_TILE, n), 0)
                 + t * _ROW_TILE)
        gt = (sr > sc) | ((sr == sc) & (jiota < iiota))
        rank = jnp.sum(jnp.where(gt, 1.0, 0.0), axis=1, keepdims=True)
        disp = (sc > 0.0) & (rank < cap)
        dispf = jnp.where(disp, 1.0, 0.0)
        siota = jax.lax.broadcasted_iota(
            jnp.int32, (_ROW_TILE, cap_pad), 1).astype(jnp.float32)
        d_ref[0, rows, :] = jnp.where((rank == siota) & disp, 1.0, 0.0)
        pd = p * dispf
        g_ref[rows, :] = pd / (pd + 1e-6)
        cnt += jnp.sum(dispf)
        psum += jnp.sum(p)
    gs_ref[0, ...] = jnp.sum(d_ref[0, ...] * g_ref[...],
                             axis=0, keepdims=True)
    f_e = cnt / n
    p_e = psum / n
    loss_ref[0, 0] += _AUX_W * e_total * f_e * p_e


def _gather_kernel(d_ref, x_ref, xe_ref):
    xe_ref[0, ...] = jax.lax.dot_general(
        d_ref[0, ...], x_ref[...], (((0,), (0,)), ((), ())),
        preferred_element_type=jnp.float32,
        precision=jax.lax.Precision.HIGHEST)


def _ffn_kernel(xe_ref, w1_ref, w2_ref, eo_ref, *, ht_total):
    ht = pl.program_id(1)
    h = jax.lax.dot_general(
        xe_ref[0, ...], w1_ref[0, ...], (((1,), (1,)), ((), ())),
        preferred_element_type=jnp.float32,
        precision=jax.lax.Precision.HIGHEST)
    h = 0.5 * h * (1.0 + jax.lax.erf(h * 0.7071067811865476))
    contrib = jax.lax.dot_general(
        h, w2_ref[0, ...], (((1,), (1,)), ((), ())),
        preferred_element_type=jnp.float32,
        precision=jax.lax.Precision.HIGHEST)

    @pl.when(ht == 0)
    def _eo_init():
        eo_ref[0, ...] = contrib

    @pl.when(ht != 0)
    def _eo_acc():
        eo_ref[0, ...] += contrib


def _scatter_kernel(d_ref, gs_ref, eo_ref, x_ref, out_ref):
    e = pl.program_id(0)
    dg = d_ref[0, ...] * gs_ref[0, ...]
    sc = jax.lax.dot_general(
        dg, eo_ref[0, ...], (((1,), (0,)), ((), ())),
        preferred_element_type=jnp.float32,
        precision=jax.lax.Precision.HIGHEST)

    @pl.when(e == 0)
    def _first():
        out_ref[...] = x_ref[...] + sc

    @pl.when(e != 0)
    def _rest():
        out_ref[...] += sc


def kernel(x, router_w, w1, w2):
    b, t, c = x.shape
    e_total, h_dim, _ = w1.shape
    n = b * t
    cap = int(1.25 * n / e_total)
    cap_pad = -(-cap // 128) * 128
    hb = 256
    ht_total = h_dim // hb

    xf = x.reshape(n, c)

    d, gs, loss = pl.pallas_call(
        functools.partial(_router_kernel, n=n, e_total=e_total, cap=cap,
                          cap_pad=cap_pad),
        grid=(e_total,),
        in_specs=[
            pl.BlockSpec((n, c), lambda e: (0, 0)),
            pl.BlockSpec((e_total, c), lambda e: (0, 0)),
        ],
        out_specs=[
            pl.BlockSpec((1, n, cap_pad), lambda e: (e, 0, 0)),
            pl.BlockSpec((1, 1, cap_pad), lambda e: (e, 0, 0)),
            pl.BlockSpec(memory_space=pltpu.SMEM),
        ],
        out_shape=[
            jax.ShapeDtypeStruct((e_total, n, cap_pad), jnp.float32),
            jax.ShapeDtypeStruct((e_total, 1, cap_pad), jnp.float32),
            jax.ShapeDtypeStruct((1, 1), jnp.float32),
        ],
        scratch_shapes=[
            pltpu.VMEM((n, e_total), jnp.float32),
            pltpu.VMEM((e_total, n), jnp.float32),
            pltpu.VMEM((n, e_total), jnp.float32),
            pltpu.VMEM((n, 1), jnp.float32),
        ],
    )(xf, router_w)

    xe = pl.pallas_call(
        _gather_kernel,
        grid=(e_total,),
        in_specs=[
            pl.BlockSpec((1, n, cap_pad), lambda e: (e, 0, 0)),
            pl.BlockSpec((n, c), lambda e: (0, 0)),
        ],
        out_specs=pl.BlockSpec((1, cap_pad, c), lambda e: (e, 0, 0)),
        out_shape=jax.ShapeDtypeStruct((e_total, cap_pad, c), jnp.float32),
    )(d, xf)

    eo = pl.pallas_call(
        functools.partial(_ffn_kernel, ht_total=ht_total),
        grid=(e_total, ht_total),
        in_specs=[
            pl.BlockSpec((1, cap_pad, c), lambda e, ht: (e, 0, 0)),
            pl.BlockSpec((1, hb, c), lambda e, ht: (e, ht, 0)),
            pl.BlockSpec((1, c, hb), lambda e, ht: (e, 0, ht)),
        ],
        out_specs=pl.BlockSpec((1, cap_pad, c), lambda e, ht: (e, 0, 0)),
        out_shape=jax.ShapeDtypeStruct((e_total, cap_pad, c), jnp.float32),
    )(xe, w1, w2)

    out = pl.pallas_call(
        _scatter_kernel,
        grid=(e_total,),
        in_specs=[
            pl.BlockSpec((1, n, cap_pad), lambda e: (e, 0, 0)),
            pl.BlockSpec((1, 1, cap_pad), lambda e: (e, 0, 0)),
            pl.BlockSpec((1, cap_pad, c), lambda e: (e, 0, 0)),
            pl.BlockSpec((n, c), lambda e: (0, 0)),
        ],
        out_specs=pl.BlockSpec((n, c), lambda e: (0, 0)),
        out_shape=jax.ShapeDtypeStruct((n, c), jnp.float32),
    )(d, gs, eo, xf)

    return out.reshape(b, t, c), loss[0, 0]


# speedup vs baseline: 3.7080x; 3.3394x over previous
import functools

import jax
import jax.numpy as jnp
from jax.experimental import pallas as pl
from jax.experimental.pallas import tpu as pltpu

_AUX_W = 0.1
_Z_W = 0.001
_ROW_TILE = 256


def _router_kernel(x_ref, rw_ref, d_ref, gs_ref, loss_ref, s_ref, st_ref,
                   probs_ref, g_ref, *, n, e_total, cap, cap_pad):
    e = pl.program_id(0)

    @pl.when(e == 0)
    def _init():
        xl = x_ref[...]
        rw = rw_ref[...]
        logits = jax.lax.dot_general(
            xl, rw, (((1,), (1,)), ((), ())),
            preferred_element_type=jnp.float32)
        m = jnp.max(logits, axis=-1, keepdims=True)
        ex = jnp.exp(logits - m)
        sume = jnp.sum(ex, axis=-1, keepdims=True)
        probs = ex / sume
        probs_ref[...] = probs
        lse = m + jnp.log(sume)
        z_loss = _Z_W * jnp.sum(lse * lse) / n
        loss_ref[0, 0] = z_loss
        pmax = jnp.max(probs, axis=-1, keepdims=True)
        eiota = jax.lax.broadcasted_iota(jnp.int32, probs.shape, 1)
        eidx = jnp.min(jnp.where(probs == pmax, eiota, e_total),
                       axis=-1, keepdims=True)
        s = jnp.where(eiota == eidx, probs, -1.0)
        s_ref[...] = s
        st_ref[...] = jnp.transpose(s)

    st = st_ref[...]
    riota = jax.lax.broadcasted_iota(jnp.int32, st.shape, 0)
    sr = jnp.sum(jnp.where(riota == e, st, 0.0), axis=0, keepdims=True)
    cnt = jnp.zeros((), jnp.float32)
    psum = jnp.zeros((), jnp.float32)
    for t in range(n // _ROW_TILE):
        rows = pl.ds(t * _ROW_TILE, _ROW_TILE)
        s_tile = s_ref[rows, :]
        p_tile = probs_ref[rows, :]
        eiota = jax.lax.broadcasted_iota(jnp.int32, s_tile.shape, 1)
        sc = jnp.sum(jnp.where(eiota == e, s_tile, 0.0),
                     axis=1, keepdims=True)
        p = jnp.sum(jnp.where(eiota == e, p_tile, 0.0),
                    axis=1, keepdims=True)
        jiota = jax.lax.broadcasted_iota(jnp.int32, (_ROW_TILE, n), 1)
        iiota = (jax.lax.broadcasted_iota(jnp.int32, (_ROW_TILE, n), 0)
                 + t * _ROW_TILE)
        gt = (sr > sc) | ((sr == sc) & (jiota < iiota))
        rank = jnp.sum(jnp.where(gt, 1.0, 0.0), axis=1, keepdims=True)
        disp = (sc > 0.0) & (rank < cap)
        dispf = jnp.where(disp, 1.0, 0.0)
        siota = jax.lax.broadcasted_iota(
            jnp.int32, (_ROW_TILE, cap_pad), 1).astype(jnp.float32)
        d_ref[0, rows, :] = jnp.where((rank == siota) & disp, 1.0, 0.0)
        pd = p * dispf
        g_ref[rows, :] = pd / (pd + 1e-6)
        cnt += jnp.sum(dispf)
        psum += jnp.sum(p)
    gs_ref[0, ...] = jnp.sum(d_ref[0, ...] * g_ref[...],
                             axis=0, keepdims=True)
    f_e = cnt / n
    p_e = psum / n
    loss_ref[0, 0] += _AUX_W * e_total * f_e * p_e


def _gather_kernel(d_ref, x_ref, xe_ref):
    xe_ref[0, ...] = jax.lax.dot_general(
        d_ref[0, ...], x_ref[...], (((0,), (0,)), ((), ())),
        preferred_element_type=jnp.float32)


def _ffn_kernel(xe_ref, w1_ref, w2_ref, eo_ref, *, ht_total):
    ht = pl.program_id(1)
    h = jax.lax.dot_general(
        xe_ref[0, ...], w1_ref[0, ...], (((1,), (1,)), ((), ())),
        preferred_element_type=jnp.float32)
    h = 0.5 * h * (1.0 + jax.lax.erf(h * 0.7071067811865476))
    contrib = jax.lax.dot_general(
        h, w2_ref[0, ...], (((1,), (1,)), ((), ())),
        preferred_element_type=jnp.float32)

    @pl.when(ht == 0)
    def _eo_init():
        eo_ref[0, ...] = contrib

    @pl.when(ht != 0)
    def _eo_acc():
        eo_ref[0, ...] += contrib


def _scatter_kernel(d_ref, gs_ref, eo_ref, x_ref, out_ref):
    e = pl.program_id(0)
    dg = d_ref[0, ...] * gs_ref[0, ...]
    sc = jax.lax.dot_general(
        dg, eo_ref[0, ...], (((1,), (0,)), ((), ())),
        preferred_element_type=jnp.float32)

    @pl.when(e == 0)
    def _first():
        out_ref[...] = x_ref[...] + sc

    @pl.when(e != 0)
    def _rest():
        out_ref[...] += sc


def kernel(x, router_w, w1, w2):
    b, t, c = x.shape
    e_total, h_dim, _ = w1.shape
    n = b * t
    cap = int(1.25 * n / e_total)
    cap_pad = -(-cap // 128) * 128
    hb = 256
    ht_total = h_dim // hb

    xf = x.reshape(n, c)

    d, gs, loss = pl.pallas_call(
        functools.partial(_router_kernel, n=n, e_total=e_total, cap=cap,
                          cap_pad=cap_pad),
        grid=(e_total,),
        in_specs=[
            pl.BlockSpec((n, c), lambda e: (0, 0)),
            pl.BlockSpec((e_total, c), lambda e: (0, 0)),
        ],
        out_specs=[
            pl.BlockSpec((1, n, cap_pad), lambda e: (e, 0, 0)),
            pl.BlockSpec((1, 1, cap_pad), lambda e: (e, 0, 0)),
            pl.BlockSpec(memory_space=pltpu.SMEM),
        ],
        out_shape=[
            jax.ShapeDtypeStruct((e_total, n, cap_pad), jnp.float32),
            jax.ShapeDtypeStruct((e_total, 1, cap_pad), jnp.float32),
            jax.ShapeDtypeStruct((1, 1), jnp.float32),
        ],
        scratch_shapes=[
            pltpu.VMEM((n, e_total), jnp.float32),
            pltpu.VMEM((e_total, n), jnp.float32),
            pltpu.VMEM((n, e_total), jnp.float32),
            pltpu.VMEM((n, 1), jnp.float32),
        ],
    )(xf, router_w)

    xe = pl.pallas_call(
        _gather_kernel,
        grid=(e_total,),
        in_specs=[
            pl.BlockSpec((1, n, cap_pad), lambda e: (e, 0, 0)),
            pl.BlockSpec((n, c), lambda e: (0, 0)),
        ],
        out_specs=pl.BlockSpec((1, cap_pad, c), lambda e: (e, 0, 0)),
        out_shape=jax.ShapeDtypeStruct((e_total, cap_pad, c), jnp.float32),
    )(d, xf)

    eo = pl.pallas_call(
        functools.partial(_ffn_kernel, ht_total=ht_total),
        grid=(e_total, ht_total),
        in_specs=[
            pl.BlockSpec((1, cap_pad, c), lambda e, ht: (e, 0, 0)),
            pl.BlockSpec((1, hb, c), lambda e, ht: (e, ht, 0)),
            pl.BlockSpec((1, c, hb), lambda e, ht: (e, 0, ht)),
        ],
        out_specs=pl.BlockSpec((1, cap_pad, c), lambda e, ht: (e, 0, 0)),
        out_shape=jax.ShapeDtypeStruct((e_total, cap_pad, c), jnp.float32),
    )(xe, w1, w2)

    out = pl.pallas_call(
        _scatter_kernel,
        grid=(e_total,),
        in_specs=[
            pl.BlockSpec((1, n, cap_pad), lambda e: (e, 0, 0)),
            pl.BlockSpec((1, 1, cap_pad), lambda e: (e, 0, 0)),
            pl.BlockSpec((1, cap_pad, c), lambda e: (e, 0, 0)),
            pl.BlockSpec((n, c), lambda e: (0, 0)),
        ],
        out_specs=pl.BlockSpec((n, c), lambda e: (0, 0)),
        out_shape=jax.ShapeDtypeStruct((n, c), jnp.float32),
    )(d, gs, eo, xf)

    return out.reshape(b, t, c), loss[0, 0]


# speedup vs baseline: 3.8767x; 1.0455x over previous
import functools

import jax
import jax.numpy as jnp
from jax.experimental import pallas as pl
from jax.experimental.pallas import tpu as pltpu

_AUX_W = 0.1
_Z_W = 0.001
_ROW_TILE = 256


def _router_kernel(x_ref, rw_ref, d_ref, gs_ref, loss_ref, s_ref, st_ref,
                   probs_ref, g_ref, *, n, e_total, cap, cap_pad):
    e = pl.program_id(0)

    @pl.when(e == 0)
    def _init():
        xl = x_ref[...]
        rw = rw_ref[...]
        logits = jax.lax.dot_general(
            xl, rw, (((1,), (1,)), ((), ())),
            preferred_element_type=jnp.float32)
        m = jnp.max(logits, axis=-1, keepdims=True)
        ex = jnp.exp(logits - m)
        sume = jnp.sum(ex, axis=-1, keepdims=True)
        probs = ex / sume
        probs_ref[...] = probs
        lse = m + jnp.log(sume)
        z_loss = _Z_W * jnp.sum(lse * lse) / n
        loss_ref[0, 0] = z_loss
        pmax = jnp.max(probs, axis=-1, keepdims=True)
        eiota = jax.lax.broadcasted_iota(jnp.int32, probs.shape, 1)
        eidx = jnp.min(jnp.where(probs == pmax, eiota, e_total),
                       axis=-1, keepdims=True)
        s = jnp.where(eiota == eidx, probs, -1.0)
        s_ref[...] = s
        st_ref[...] = jnp.transpose(s)

    st = st_ref[...]
    riota = jax.lax.broadcasted_iota(jnp.int32, st.shape, 0)
    sr = jnp.sum(jnp.where(riota == e, st, 0.0), axis=0, keepdims=True)
    cnt = jnp.zeros((), jnp.float32)
    psum = jnp.zeros((), jnp.float32)
    for t in range(n // _ROW_TILE):
        rows = pl.ds(t * _ROW_TILE, _ROW_TILE)
        s_tile = s_ref[rows, :]
        p_tile = probs_ref[rows, :]
        eiota = jax.lax.broadcasted_iota(jnp.int32, s_tile.shape, 1)
        sc = jnp.sum(jnp.where(eiota == e, s_tile, 0.0),
                     axis=1, keepdims=True)
        p = jnp.sum(jnp.where(eiota == e, p_tile, 0.0),
                    axis=1, keepdims=True)
        jiota = jax.lax.broadcasted_iota(jnp.int32, (_ROW_TILE, n), 1)
        iiota = (jax.lax.broadcasted_iota(jnp.int32, (_ROW_TILE, n), 0)
                 + t * _ROW_TILE)
        gt = (sr > sc) | ((sr == sc) & (jiota < iiota))
        rank = jnp.sum(jnp.where(gt, 1.0, 0.0), axis=1, keepdims=True)
        disp = (sc > 0.0) & (rank < cap)
        dispf = jnp.where(disp, 1.0, 0.0)
        siota = jax.lax.broadcasted_iota(
            jnp.int32, (_ROW_TILE, cap_pad), 1).astype(jnp.float32)
        d_ref[0, rows, :] = jnp.where((rank == siota) & disp, 1.0, 0.0)
        pd = p * dispf
        g_ref[rows, :] = pd / (pd + 1e-6)
        cnt += jnp.sum(dispf)
        psum += jnp.sum(p)
    gs_ref[0, ...] = jnp.sum(d_ref[0, ...] * g_ref[...],
                             axis=0, keepdims=True)
    f_e = cnt / n
    p_e = psum / n
    loss_ref[0, 0] += _AUX_W * e_total * f_e * p_e


def _gather_kernel(d_ref, x_ref, xe_ref):
    xe_ref[0, ...] = jax.lax.dot_general(
        d_ref[0, ...], x_ref[...], (((0,), (0,)), ((), ())),
        preferred_element_type=jnp.float32)


def _ffn_kernel(xe_ref, w1_ref, w2_ref, eo_ref, *, ht_total):
    ht = pl.program_id(1)
    h = jax.lax.dot_general(
        xe_ref[0, ...], w1_ref[0, ...], (((1,), (1,)), ((), ())),
        preferred_element_type=jnp.float32)
    h = 0.5 * h * (1.0 + jax.lax.erf(h * 0.7071067811865476))
    contrib = jax.lax.dot_general(
        h, w2_ref[0, ...], (((1,), (1,)), ((), ())),
        preferred_element_type=jnp.float32)

    @pl.when(ht == 0)
    def _eo_init():
        eo_ref[0, ...] = contrib

    @pl.when(ht != 0)
    def _eo_acc():
        eo_ref[0, ...] += contrib


def _scatter_kernel(d_ref, gs_ref, eo_ref, x_ref, out_ref):
    e = pl.program_id(0)
    dg = d_ref[0, ...] * gs_ref[0, ...]
    sc = jax.lax.dot_general(
        dg, eo_ref[0, ...], (((1,), (0,)), ((), ())),
        preferred_element_type=jnp.float32)

    @pl.when(e == 0)
    def _first():
        out_ref[...] = x_ref[...] + sc

    @pl.when(e != 0)
    def _rest():
        out_ref[...] += sc


def kernel(x, router_w, w1, w2):
    b, t, c = x.shape
    e_total, h_dim, _ = w1.shape
    n = b * t
    cap = int(1.25 * n / e_total)
    cap_pad = cap
    hb = 256
    ht_total = h_dim // hb

    xf = x.reshape(n, c)

    d, gs, loss = pl.pallas_call(
        functools.partial(_router_kernel, n=n, e_total=e_total, cap=cap,
                          cap_pad=cap_pad),
        grid=(e_total,),
        in_specs=[
            pl.BlockSpec((n, c), lambda e: (0, 0)),
            pl.BlockSpec((e_total, c), lambda e: (0, 0)),
        ],
        out_specs=[
            pl.BlockSpec((1, n, cap_pad), lambda e: (e, 0, 0)),
            pl.BlockSpec((1, 1, cap_pad), lambda e: (e, 0, 0)),
            pl.BlockSpec(memory_space=pltpu.SMEM),
        ],
        out_shape=[
            jax.ShapeDtypeStruct((e_total, n, cap_pad), jnp.float32),
            jax.ShapeDtypeStruct((e_total, 1, cap_pad), jnp.float32),
            jax.ShapeDtypeStruct((1, 1), jnp.float32),
        ],
        scratch_shapes=[
            pltpu.VMEM((n, e_total), jnp.float32),
            pltpu.VMEM((e_total, n), jnp.float32),
            pltpu.VMEM((n, e_total), jnp.float32),
            pltpu.VMEM((n, 1), jnp.float32),
        ],
    )(xf, router_w)

    xe = pl.pallas_call(
        _gather_kernel,
        grid=(e_total,),
        in_specs=[
            pl.BlockSpec((1, n, cap_pad), lambda e: (e, 0, 0)),
            pl.BlockSpec((n, c), lambda e: (0, 0)),
        ],
        out_specs=pl.BlockSpec((1, cap_pad, c), lambda e: (e, 0, 0)),
        out_shape=jax.ShapeDtypeStruct((e_total, cap_pad, c), jnp.float32),
    )(d, xf)

    eo = pl.pallas_call(
        functools.partial(_ffn_kernel, ht_total=ht_total),
        grid=(e_total, ht_total),
        in_specs=[
            pl.BlockSpec((1, cap_pad, c), lambda e, ht: (e, 0, 0)),
            pl.BlockSpec((1, hb, c), lambda e, ht: (e, ht, 0)),
            pl.BlockSpec((1, c, hb), lambda e, ht: (e, 0, ht)),
        ],
        out_specs=pl.BlockSpec((1, cap_pad, c), lambda e, ht: (e, 0, 0)),
        out_shape=jax.ShapeDtypeStruct((e_total, cap_pad, c), jnp.float32),
    )(xe, w1, w2)

    out = pl.pallas_call(
        _scatter_kernel,
        grid=(e_total,),
        in_specs=[
            pl.BlockSpec((1, n, cap_pad), lambda e: (e, 0, 0)),
            pl.BlockSpec((1, 1, cap_pad), lambda e: (e, 0, 0)),
            pl.BlockSpec((1, cap_pad, c), lambda e: (e, 0, 0)),
            pl.BlockSpec((n, c), lambda e: (0, 0)),
        ],
        out_specs=pl.BlockSpec((n, c), lambda e: (0, 0)),
        out_shape=jax.ShapeDtypeStruct((n, c), jnp.float32),
    )(d, gs, eo, xf)

    return out.reshape(b, t, c), loss[0, 0]
